# baseline (speedup 1.0000x reference)
import functools

import jax
import jax.numpy as jnp
from jax import lax
from jax.experimental import pallas as pl

IMG_H = 224
IMG_W = 224
N_PTS = 128
MIN_DEPTH = 0.1
MAX_DEPTH = 8.0

H_BLK = 16
N_BLOCKS = IMG_H // H_BLK
PIX_BLK = H_BLK * IMG_W


def _rays_kernel(R_ref, T_ref, f_ref, pp_ref, xs3_ref, xs2_ref, ys_ref, d_ref,
                 orig_ref, dir_ref, len_ref, xy_ref):
    fx = f_ref[0, 0, 0]
    fy = f_ref[0, 0, 1]
    px = pp_ref[0, 0, 0]
    py = pp_ref[0, 0, 1]

    r00 = R_ref[0, 0, 0]
    r01 = R_ref[0, 0, 1]
    r02 = R_ref[0, 0, 2]
    r10 = R_ref[0, 1, 0]
    r11 = R_ref[0, 1, 1]
    r12 = R_ref[0, 1, 2]
    r20 = R_ref[0, 2, 0]
    r21 = R_ref[0, 2, 1]
    r22 = R_ref[0, 2, 2]

    dxr = (xs3_ref[...] - px) / fx
    dyc = (ys_ref[...] - py) / fy

    m3 = lax.broadcasted_iota(jnp.int32, (1, 3 * IMG_W), 1) % 3
    is0 = m3 == 0
    is1 = m3 == 1
    row_a = jnp.where(is0, r00, jnp.where(is1, r10, r20))
    row_b = jnp.where(is0, r01, jnp.where(is1, r11, r21))
    row_c = jnp.where(is0, r02, jnp.where(is1, r12, r22))

    def _bf(x):
        return x.astype(jnp.bfloat16).astype(jnp.float32)

    dxb = _bf(dxr)
    dyb = _bf(dyc)
    num = dxb * _bf(row_a) + (dyb * _bf(row_b) + _bf(row_c))
    den = jnp.abs(dxb * _bf(r00) + (dyb * _bf(r01) + _bf(r02)))
    dir_ref[0] = num / den

    t0 = T_ref[0, 0, 0]
    t1 = T_ref[0, 0, 1]
    t2 = T_ref[0, 0, 2]
    c0 = -(r00 * t0 + r01 * t1 + r02 * t2)
    c1 = -(r10 * t0 + r11 * t1 + r12 * t2)
    c2 = -(r20 * t0 + r21 * t1 + r22 * t2)
    row_ctr = jnp.where(is0, c0, jnp.where(is1, c1, c2))
    orig_ref[0] = jnp.broadcast_to(row_ctr, (H_BLK, 3 * IMG_W))

    m2 = lax.broadcasted_iota(jnp.int32, (1, 2 * IMG_W), 1) % 2
    xy_ref[0] = jnp.where(m2 == 0, xs2_ref[...], ys_ref[...])

    len_ref[0] = jnp.broadcast_to(d_ref[...], (PIX_BLK, N_PTS))


@jax.jit
def kernel(R, T, focal_length, principal_point):
    B = R.shape[0]
    xs = jnp.linspace(-1.0, 1.0, IMG_W, dtype=jnp.float32)
    ys = jnp.linspace(-1.0, 1.0, IMG_H, dtype=jnp.float32)
    depths = jnp.linspace(MIN_DEPTH, MAX_DEPTH, N_PTS, dtype=jnp.float32)
    xs3 = jnp.repeat(xs, 3)[None, :]
    xs2 = jnp.repeat(xs, 2)[None, :]
    ys_col = ys[:, None]
    depths_row = depths[None, :]

    n_per = IMG_H * IMG_W
    grid = (B, N_BLOCKS)
    out_shapes = (
        jax.ShapeDtypeStruct((B, IMG_H, 3 * IMG_W), jnp.float32),
        jax.ShapeDtypeStruct((B, IMG_H, 3 * IMG_W), jnp.float32),
        jax.ShapeDtypeStruct((B, n_per, N_PTS), jnp.float32),
        jax.ShapeDtypeStruct((B, IMG_H, 2 * IMG_W), jnp.float32),
    )
    in_specs = [
        pl.BlockSpec((1, 3, 3), lambda b, n: (b, 0, 0)),
        pl.BlockSpec((1, 1, 3), lambda b, n: (b, 0, 0)),
        pl.BlockSpec((1, 1, 2), lambda b, n: (b, 0, 0)),
        pl.BlockSpec((1, 1, 2), lambda b, n: (b, 0, 0)),
        pl.BlockSpec((1, 3 * IMG_W), lambda b, n: (0, 0)),
        pl.BlockSpec((1, 2 * IMG_W), lambda b, n: (0, 0)),
        pl.BlockSpec((H_BLK, 1), lambda b, n: (n, 0)),
        pl.BlockSpec((1, N_PTS), lambda b, n: (0, 0)),
    ]
    out_specs = (
        pl.BlockSpec((1, H_BLK, 3 * IMG_W), lambda b, n: (b, n, 0)),
        pl.BlockSpec((1, H_BLK, 3 * IMG_W), lambda b, n: (b, n, 0)),
        pl.BlockSpec((1, PIX_BLK, N_PTS), lambda b, n: (b, n, 0)),
        pl.BlockSpec((1, H_BLK, 2 * IMG_W), lambda b, n: (b, n, 0)),
    )
    origins, directions, lengths, xys = pl.pallas_call(
        _rays_kernel,
        grid=grid,
        in_specs=in_specs,
        out_specs=out_specs,
        out_shape=out_shapes,
    )(R, T[:, None, :], focal_length[:, None, :], principal_point[:, None, :],
      xs3, xs2, ys_col, depths_row)

    origins = origins.reshape(B, n_per, 3)
    directions = directions.reshape(B, n_per, 3)
    xys = xys.reshape(B, n_per, 2)
    return (origins, directions, lengths, xys)


# speedup vs baseline: 11.4065x; 11.4065x over previous
import jax
import jax.numpy as jnp
from jax import lax
from jax.experimental import pallas as pl

IMG_H = 224
IMG_W = 224
N_PTS = 128
MIN_DEPTH = 0.1
MAX_DEPTH = 8.0
B = 4

N_PER = IMG_H * IMG_W
N_CHUNKS = N_PER // 128
GRID = 28
DIR_ROWS = (N_CHUNKS * B) // GRID
XY_ROWS = (N_CHUNKS * 2) // (GRID // B)
LEN_ROWS = N_PER // (GRID // B)


def _f32(x):
    return x.astype(jnp.float32)


def _bf(x):
    return x.astype(jnp.bfloat16).astype(jnp.float32)


def _ndc(idx, last):
    t = _f32(idx) / jnp.float32(last)
    return jnp.where(idx == last, jnp.float32(1.0), t - (jnp.float32(1.0) - t))


def _pix_to_hw(npix):
    m = npix >> 5
    h = (m * 9363) >> 16
    w = npix - h * 224
    return h, w


def _rays_kernel(R_ref, T_ref, f_ref, pp_ref, d_ref,
                 orig_ref, dir_ref, len_ref, xy_ref):
    s = pl.program_id(0)
    lane = lax.broadcasted_iota(jnp.int32, (1, 128), 1)

    def sel_b(bb, vals):
        return jnp.where(bb == 0, vals[0],
                         jnp.where(bb == 1, vals[1],
                                   jnp.where(bb == 2, vals[2], vals[3])))

    r = lax.broadcasted_iota(jnp.int32, (DIR_ROWS, 1), 0) + DIR_ROWS * s
    k = r >> 2
    bb = r & 3
    npix = (k << 7) + lane
    h, w = _pix_to_hw(npix)

    fx = sel_b(bb, [f_ref[i, 0, 0] for i in range(B)])
    fy = sel_b(bb, [f_ref[i, 0, 1] for i in range(B)])
    px = sel_b(bb, [pp_ref[i, 0, 0] for i in range(B)])
    py = sel_b(bb, [pp_ref[i, 0, 1] for i in range(B)])

    dx = (_ndc(w, 223) - px) / fx
    dy = (_ndc(h, 223) - py) / fy
    dxb = _bf(dx)
    dyb = _bf(dy)

    rows = [[sel_b(bb, [R_ref[i, ci, j] for i in range(B)])
             for j in range(3)] for ci in range(3)]

    comp = lax.broadcasted_iota(jnp.int32, (3, 1, 1), 0)

    def sel_c(vals):
        return jnp.where(comp == 0, vals[0][None],
                         jnp.where(comp == 1, vals[1][None], vals[2][None]))

    ra = sel_c([rows[0][0], rows[1][0], rows[2][0]])
    rb = sel_c([rows[0][1], rows[1][1], rows[2][1]])
    rc = sel_c([rows[0][2], rows[1][2], rows[2][2]])
    num = dxb[None] * _bf(ra) + (dyb[None] * _bf(rb) + _bf(rc))
    den = jnp.abs(dxb * _bf(rows[0][0]) + (dyb * _bf(rows[0][1])
                                           + _bf(rows[0][2])))
    dir_ref[...] = num / den[None]

    t = [[T_ref[i, 0, j] for j in range(3)] for i in range(B)]
    rr = [[[R_ref[i, ci, j] for j in range(3)] for ci in range(3)]
          for i in range(B)]
    ctr = [[-(rr[i][ci][0] * t[i][0] + rr[i][ci][1] * t[i][1]
              + rr[i][ci][2] * t[i][2]) for i in range(B)] for ci in range(3)]
    ctr_col = sel_c([sel_b(bb, ctr[0]), sel_b(bb, ctr[1]), sel_b(bb, ctr[2])])
    orig_ref[...] = jnp.broadcast_to(ctr_col, (3, DIR_ROWS, 128))

    r2 = lax.broadcasted_iota(jnp.int32, (XY_ROWS, 1), 0) + XY_ROWS * (s % 7)
    c2 = r2 & 1
    npix2 = ((r2 >> 1) << 7) + lane
    h2, w2 = _pix_to_hw(npix2)
    xy_ref[0] = jnp.where(c2 == 0, _ndc(w2, 223), _ndc(h2, 223))

    len_ref[0] = jnp.broadcast_to(d_ref[...], (LEN_ROWS, N_PTS))


@jax.jit
def kernel(R, T, focal_length, principal_point):
    depths_row = jnp.linspace(MIN_DEPTH, MAX_DEPTH, N_PTS,
                              dtype=jnp.float32)[None, :]
    out_shapes = (
        jax.ShapeDtypeStruct((3, B * N_CHUNKS, 128), jnp.float32),
        jax.ShapeDtypeStruct((3, B * N_CHUNKS, 128), jnp.float32),
        jax.ShapeDtypeStruct((B, N_PER, N_PTS), jnp.float32),
        jax.ShapeDtypeStruct((B, 2 * N_CHUNKS, 128), jnp.float32),
    )
    in_specs = [
        pl.BlockSpec((B, 3, 3), lambda s: (0, 0, 0)),
        pl.BlockSpec((B, 1, 3), lambda s: (0, 0, 0)),
        pl.BlockSpec((B, 1, 2), lambda s: (0, 0, 0)),
        pl.BlockSpec((B, 1, 2), lambda s: (0, 0, 0)),
        pl.BlockSpec((1, N_PTS), lambda s: (0, 0)),
    ]
    out_specs = (
        pl.BlockSpec((3, DIR_ROWS, 128), lambda s: (0, s, 0)),
        pl.BlockSpec((3, DIR_ROWS, 128), lambda s: (0, s, 0)),
        pl.BlockSpec((1, LEN_ROWS, N_PTS), lambda s: (s // 7, s % 7, 0)),
        pl.BlockSpec((1, XY_ROWS, 128), lambda s: (s // 7, s % 7, 0)),
    )
    origins_p, directions_p, lengths, xys_p = pl.pallas_call(
        _rays_kernel,
        grid=(GRID,),
        in_specs=in_specs,
        out_specs=out_specs,
        out_shape=out_shapes,
    )(R, T[:, None, :], focal_length[:, None, :], principal_point[:, None, :],
      depths_row)

    origins = origins_p.reshape(3, N_CHUNKS, B, 128).transpose(2, 1, 3, 0)
    origins = origins.reshape(B, N_PER, 3)
    directions = directions_p.reshape(3, N_CHUNKS, B, 128).transpose(2, 1, 3, 0)
    directions = directions.reshape(B, N_PER, 3)
    xys = xys_p.reshape(B, N_CHUNKS, 2, 128).transpose(0, 1, 3, 2)
    xys = xys.reshape(B, N_PER, 2)
    return (origins, directions, lengths, xys)


# speedup vs baseline: 11.6643x; 1.0226x over previous
import jax
import jax.numpy as jnp
from jax import lax
from jax.experimental import pallas as pl

IMG_H = 224
IMG_W = 224
N_PTS = 128
MIN_DEPTH = 0.1
MAX_DEPTH = 8.0
B = 4

N_PER = IMG_H * IMG_W
N_CHUNKS = N_PER // 128
GRID = 28
DIR_ROWS = (N_CHUNKS * B) // GRID
XY_ROWS = (N_CHUNKS * 2) // (GRID // B)
LEN_ROWS = N_PER // (GRID // B)


def _f32(x):
    return x.astype(jnp.float32)


def _bf(x):
    return x.astype(jnp.bfloat16).astype(jnp.float32)


def _ndc(idx, last):
    t = _f32(idx) / jnp.float32(last)
    return jnp.where(idx == last, jnp.float32(1.0), t - (jnp.float32(1.0) - t))


def _pix_to_hw(npix):
    m = npix >> 5
    h = (m * 9363) >> 16
    w = npix - h * 224
    return h, w


def _rays_kernel(P_ref, orig_ref, dir_ref, len_ref, xy_ref):
    s = pl.program_id(0)
    lane = lax.broadcasted_iota(jnp.int32, (1, 128), 1)

    def sel_b(bb, vals):
        return jnp.where(bb == 0, vals[0],
                         jnp.where(bb == 1, vals[1],
                                   jnp.where(bb == 2, vals[2], vals[3])))

    r = lax.broadcasted_iota(jnp.int32, (DIR_ROWS, 1), 0) + DIR_ROWS * s
    k = r >> 2
    bb = r & 3
    npix = (k << 7) + lane
    h, w = _pix_to_hw(npix)

    fx = sel_b(bb, [P_ref[i, 12] for i in range(B)])
    fy = sel_b(bb, [P_ref[i, 13] for i in range(B)])
    px = sel_b(bb, [P_ref[i, 14] for i in range(B)])
    py = sel_b(bb, [P_ref[i, 15] for i in range(B)])

    dx = (_ndc(w, 223) - px) / fx
    dy = (_ndc(h, 223) - py) / fy
    dxb = _bf(dx)
    dyb = _bf(dy)

    rows = [[sel_b(bb, [P_ref[i, 3 * ci + j] for i in range(B)])
             for j in range(3)] for ci in range(3)]

    comp = lax.broadcasted_iota(jnp.int32, (3, 1, 1), 0)

    def sel_c(vals):
        return jnp.where(comp == 0, vals[0][None],
                         jnp.where(comp == 1, vals[1][None], vals[2][None]))

    ra = sel_c([rows[0][0], rows[1][0], rows[2][0]])
    rb = sel_c([rows[0][1], rows[1][1], rows[2][1]])
    rc = sel_c([rows[0][2], rows[1][2], rows[2][2]])
    num = dxb[None] * _bf(ra) + (dyb[None] * _bf(rb) + _bf(rc))
    den = jnp.abs(dxb * _bf(rows[0][0]) + (dyb * _bf(rows[0][1])
                                           + _bf(rows[0][2])))
    dir_ref[...] = num / den[None]

    t = [[P_ref[i, 9 + j] for j in range(3)] for i in range(B)]
    rr = [[[P_ref[i, 3 * ci + j] for j in range(3)] for ci in range(3)]
          for i in range(B)]
    ctr = [[-(rr[i][ci][0] * t[i][0] + rr[i][ci][1] * t[i][1]
              + rr[i][ci][2] * t[i][2]) for i in range(B)] for ci in range(3)]
    ctr_col = sel_c([sel_b(bb, ctr[0]), sel_b(bb, ctr[1]), sel_b(bb, ctr[2])])
    orig_ref[...] = jnp.broadcast_to(ctr_col, (3, DIR_ROWS, 128))

    r2 = lax.broadcasted_iota(jnp.int32, (XY_ROWS, 1), 0) + XY_ROWS * (s % 7)
    c2 = r2 & 1
    npix2 = ((r2 >> 1) << 7) + lane
    h2, w2 = _pix_to_hw(npix2)
    xy_ref[0] = jnp.where(c2 == 0, _ndc(w2, 223), _ndc(h2, 223))

    dlane = lax.broadcasted_iota(jnp.int32, (1, N_PTS), 1)
    td = _f32(dlane) / jnp.float32(N_PTS - 1)
    depths = jnp.where(dlane == N_PTS - 1, jnp.float32(MAX_DEPTH),
                       jnp.float32(MIN_DEPTH) * (jnp.float32(1.0) - td)
                       + jnp.float32(MAX_DEPTH) * td)
    len_ref[0] = jnp.broadcast_to(depths, (LEN_ROWS, N_PTS))


@jax.jit
def kernel(R, T, focal_length, principal_point):
    params = jnp.concatenate(
        [R.reshape(B, 9), T, focal_length, principal_point], axis=1)
    out_shapes = (
        jax.ShapeDtypeStruct((3, B * N_CHUNKS, 128), jnp.float32),
        jax.ShapeDtypeStruct((3, B * N_CHUNKS, 128), jnp.float32),
        jax.ShapeDtypeStruct((B, N_PER, N_PTS), jnp.float32),
        jax.ShapeDtypeStruct((B, 2 * N_CHUNKS, 128), jnp.float32),
    )
    in_specs = [
        pl.BlockSpec((B, 16), lambda s: (0, 0)),
    ]
    out_specs = (
        pl.BlockSpec((3, DIR_ROWS, 128), lambda s: (0, s, 0)),
        pl.BlockSpec((3, DIR_ROWS, 128), lambda s: (0, s, 0)),
        pl.BlockSpec((1, LEN_ROWS, N_PTS), lambda s: (s // 7, s % 7, 0)),
        pl.BlockSpec((1, XY_ROWS, 128), lambda s: (s // 7, s % 7, 0)),
    )
    origins_p, directions_p, lengths, xys_p = pl.pallas_call(
        _rays_kernel,
        grid=(GRID,),
        in_specs=in_specs,
        out_specs=out_specs,
        out_shape=out_shapes,
    )(params)

    origins = origins_p.reshape(3, N_CHUNKS, B, 128).transpose(2, 1, 3, 0)
    origins = origins.reshape(B, N_PER, 3)
    directions = directions_p.reshape(3, N_CHUNKS, B, 128).transpose(2, 1, 3, 0)
    directions = directions.reshape(B, N_PER, 3)
    xys = xys_p.reshape(B, N_CHUNKS, 2, 128).transpose(0, 1, 3, 2)
    xys = xys.reshape(B, N_PER, 2)
    return (origins, directions, lengths, xys)


# speedup vs baseline: 12.7349x; 1.0918x over previous
import jax
import jax.numpy as jnp
from jax import lax
from jax.experimental import pallas as pl

IMG_H = 224
IMG_W = 224
N_PTS = 128
MIN_DEPTH = 0.1
MAX_DEPTH = 8.0
B = 4

N_PER = IMG_H * IMG_W
N_CHUNKS = N_PER // 128
GRID = 28
DIR_ROWS = (N_CHUNKS * B) // GRID
XY_ROWS = (N_CHUNKS * 2) // (GRID // B)
LEN_ROWS = N_PER // (GRID // B)


def _f32(x):
    return x.astype(jnp.float32)


def _bf(x):
    return x.astype(jnp.bfloat16).astype(jnp.float32)


def _ndc(idx, last):
    t = _f32(idx) / jnp.float32(last)
    return jnp.where(idx == last, jnp.float32(1.0), t - (jnp.float32(1.0) - t))


def _pix_to_hw(npix):
    m = npix >> 5
    h = (m * 9363) >> 16
    w = npix - h * 224
    return h, w


def _rays_kernel(R_ref, T_ref, f_ref, pp_ref, orig_ref, dir_ref, len_ref, xy_ref):
    s = pl.program_id(0)
    lane = lax.broadcasted_iota(jnp.int32, (1, 128), 1)

    def sel_b(bb, vals):
        return jnp.where(bb == 0, vals[0],
                         jnp.where(bb == 1, vals[1],
                                   jnp.where(bb == 2, vals[2], vals[3])))

    r = lax.broadcasted_iota(jnp.int32, (DIR_ROWS, 1), 0) + DIR_ROWS * s
    k = r >> 2
    bb = r & 3
    npix = (k << 7) + lane
    h, w = _pix_to_hw(npix)

    fx = sel_b(bb, [f_ref[0, i] for i in range(B)])
    fy = sel_b(bb, [f_ref[1, i] for i in range(B)])
    px = sel_b(bb, [pp_ref[0, i] for i in range(B)])
    py = sel_b(bb, [pp_ref[1, i] for i in range(B)])

    dx = (_ndc(w, 223) - px) / fx
    dy = (_ndc(h, 223) - py) / fy
    dxb = _bf(dx)
    dyb = _bf(dy)

    rows = [[sel_b(bb, [R_ref[ci, i, j] for i in range(B)])
             for j in range(3)] for ci in range(3)]

    comp = lax.broadcasted_iota(jnp.int32, (3, 1, 1), 0)

    def sel_c(vals):
        return jnp.where(comp == 0, vals[0][None],
                         jnp.where(comp == 1, vals[1][None], vals[2][None]))

    ra = sel_c([rows[0][0], rows[1][0], rows[2][0]])
    rb = sel_c([rows[0][1], rows[1][1], rows[2][1]])
    rc = sel_c([rows[0][2], rows[1][2], rows[2][2]])
    num = dxb[None] * _bf(ra) + (dyb[None] * _bf(rb) + _bf(rc))
    den = jnp.abs(dxb * _bf(rows[0][0]) + (dyb * _bf(rows[0][1])
                                           + _bf(rows[0][2])))
    dir_ref[...] = num / den[None]

    t = [[T_ref[i, j] for j in range(3)] for i in range(B)]
    rr = [[[R_ref[ci, i, j] for j in range(3)] for ci in range(3)]
          for i in range(B)]
    ctr = [[-(rr[i][ci][0] * t[i][0] + rr[i][ci][1] * t[i][1]
              + rr[i][ci][2] * t[i][2]) for i in range(B)] for ci in range(3)]
    ctr_col = sel_c([sel_b(bb, ctr[0]), sel_b(bb, ctr[1]), sel_b(bb, ctr[2])])
    orig_ref[...] = jnp.broadcast_to(ctr_col, (3, DIR_ROWS, 128))

    r2 = lax.broadcasted_iota(jnp.int32, (XY_ROWS, 1), 0) + XY_ROWS * (s % 7)
    c2 = r2 & 1
    npix2 = ((r2 >> 1) << 7) + lane
    h2, w2 = _pix_to_hw(npix2)
    xy_ref[0] = jnp.where(c2 == 0, _ndc(w2, 223), _ndc(h2, 223))

    dlane = lax.broadcasted_iota(jnp.int32, (1, N_PTS), 1)
    td = _f32(dlane) / jnp.float32(N_PTS - 1)
    depths = jnp.where(dlane == N_PTS - 1, jnp.float32(MAX_DEPTH),
                       jnp.float32(MIN_DEPTH) * (jnp.float32(1.0) - td)
                       + jnp.float32(MAX_DEPTH) * td)
    len_ref[0] = jnp.broadcast_to(depths, (LEN_ROWS, N_PTS))


@jax.jit
def kernel(R, T, focal_length, principal_point):
    out_shapes = (
        jax.ShapeDtypeStruct((3, B * N_CHUNKS, 128), jnp.float32),
        jax.ShapeDtypeStruct((3, B * N_CHUNKS, 128), jnp.float32),
        jax.ShapeDtypeStruct((B, N_PER, N_PTS), jnp.float32),
        jax.ShapeDtypeStruct((B, 2 * N_CHUNKS, 128), jnp.float32),
    )
    in_specs = [
        pl.BlockSpec((3, B, 3), lambda s: (0, 0, 0)),
        pl.BlockSpec((B, 3), lambda s: (0, 0)),
        pl.BlockSpec((2, B), lambda s: (0, 0)),
        pl.BlockSpec((2, B), lambda s: (0, 0)),
    ]
    out_specs = (
        pl.BlockSpec((3, DIR_ROWS, 128), lambda s: (0, s, 0)),
        pl.BlockSpec((3, DIR_ROWS, 128), lambda s: (0, s, 0)),
        pl.BlockSpec((1, LEN_ROWS, N_PTS), lambda s: (s // 7, s % 7, 0)),
        pl.BlockSpec((1, XY_ROWS, 128), lambda s: (s // 7, s % 7, 0)),
    )
    origins_p, directions_p, lengths, xys_p = pl.pallas_call(
        _rays_kernel,
        grid=(GRID,),
        in_specs=in_specs,
        out_specs=out_specs,
        out_shape=out_shapes,
    )(R.transpose(1, 0, 2), T, focal_length.T, principal_point.T)

    origins = origins_p.reshape(3, N_CHUNKS, B, 128).transpose(2, 1, 3, 0)
    origins = origins.reshape(B, N_PER, 3)
    directions = directions_p.reshape(3, N_CHUNKS, B, 128).transpose(2, 1, 3, 0)
    directions = directions.reshape(B, N_PER, 3)
    xys = xys_p.reshape(B, N_CHUNKS, 2, 128).transpose(0, 1, 3, 2)
    xys = xys.reshape(B, N_PER, 2)
    return (origins, directions, lengths, xys)
